# baseline (speedup 1.0000x reference)
import functools

import numpy as np
import jax
import jax.numpy as jnp
from jax import lax
from jax.experimental import pallas as pl
from jax.experimental.pallas import tpu as pltpu

_H = 32
_W = 32
_CIN = 128
_COUT = 256
_KH = 3
_KW = 3
_HW = _H * _W
_K = _KH * _KW * _CIN
_B = 4


def _build_im2col(xb, xc_ref, slot):
    w_idx = lax.rem(lax.broadcasted_iota(jnp.int32, (_B * _HW, 1), 0), _W)
    xl = jnp.where(w_idx == _W - 1, jnp.bfloat16(0), xb)
    xr = jnp.where(w_idx == 0, jnp.bfloat16(0), xb)
    taps = (xl, xb, xr)

    for b in range(_B):
        base = b * _HW
        zeros = jnp.zeros((48, _K), jnp.bfloat16)
        xc_ref[slot, pl.ds(base, 48), :] = zeros
        xc_ref[slot, pl.ds(base + _HW - 48, 48), :] = zeros
        for dy in range(_KH):
            for dx in range(_KW):
                off = (dy - 1) * _W + (dx - 1)
                lo = max(0, -off)
                hi = min(_HW, _HW - off)
                k0 = (dy * _KW + dx) * _CIN
                xc_ref[slot, pl.ds(base + lo, hi - lo), k0:k0 + _CIN] = (
                    taps[dx][base + lo + off:base + hi + off])


def _conv_pipeline(x_hbm, w_ref, o_hbm, x_buf, o_buf, xc_ref, in_sem, out_sem,
                   *, n_steps):
    def dma_in(slot, step):
        return pltpu.make_async_copy(
            x_hbm.at[pl.ds(step * _B, _B)], x_buf.at[slot], in_sem.at[slot])

    def dma_out(slot, step):
        return pltpu.make_async_copy(
            o_buf.at[slot], o_hbm.at[pl.ds(step * _B, _B)], out_sem.at[slot])

    dma_in(0, 0).start()

    for i in range(n_steps + 1):
        if i < n_steps:
            if i + 1 < n_steps:
                dma_in((i + 1) % 2, i + 1).start()
            dma_in(i % 2, i).wait()
            xb = x_buf[i % 2].reshape(_B * _HW, _CIN)
            _build_im2col(xb, xc_ref, i % 2)
        if i >= 1:
            j = i - 1
            js = j % 2
            if j >= 2:
                dma_out(js, j - 2).wait()
            p = jnp.dot(xc_ref[js], w_ref[...],
                        preferred_element_type=jnp.float32)
            o_buf[js] = p.reshape(_B, _HW, _COUT)
            dma_out(js, j).start()

    dma_out((n_steps - 2) % 2, n_steps - 2).wait()
    dma_out((n_steps - 1) % 2, n_steps - 1).wait()


def kernel(x, weight):
    n = x.shape[0]
    n_steps = n // _B
    scale = 1.0 / float(np.sqrt(np.prod(weight.shape[1:])))
    w_t = jnp.transpose(weight, (2, 3, 1, 0)).reshape(_K, _COUT)
    w_t = (w_t * scale).astype(jnp.bfloat16)
    x_nhwc = jnp.transpose(x, (0, 2, 3, 1)).reshape(n, _HW, _CIN)
    x_nhwc = x_nhwc.astype(jnp.bfloat16)

    body = functools.partial(_conv_pipeline, n_steps=n_steps)
    out = pl.pallas_call(
        body,
        out_shape=jax.ShapeDtypeStruct((n, _HW, _COUT), jnp.float32),
        in_specs=[
            pl.BlockSpec(memory_space=pltpu.MemorySpace.HBM),
            pl.BlockSpec(memory_space=pltpu.MemorySpace.VMEM),
        ],
        out_specs=pl.BlockSpec(memory_space=pltpu.MemorySpace.HBM),
        scratch_shapes=[
            pltpu.VMEM((2, _B, _HW, _CIN), jnp.bfloat16),
            pltpu.VMEM((2, _B, _HW, _COUT), jnp.float32),
            pltpu.VMEM((2, _B * _HW, _K), jnp.bfloat16),
            pltpu.SemaphoreType.DMA((2,)),
            pltpu.SemaphoreType.DMA((2,)),
        ],
        compiler_params=pltpu.CompilerParams(
            vmem_limit_bytes=64 * 1024 * 1024),
    )(x_nhwc, w_t)
    out = out.reshape(n, _H, _W, _COUT)
    return jnp.transpose(out, (0, 3, 1, 2))


# speedup vs baseline: 1.4090x; 1.4090x over previous
import numpy as np
import jax
import jax.numpy as jnp
from jax import lax
from jax.experimental import pallas as pl
from jax.experimental.pallas import tpu as pltpu

_H = 32
_W = 32
_CIN = 128
_COUT = 256
_KH = 3
_KW = 3
_HW = _H * _W
_K = _KH * _KW * _CIN
_B = 4


def _conv_body(x_ref, w_ref, o_ref, xc_ref):
    i = pl.program_id(0)
    s = lax.rem(i, 2)

    xb = x_ref[...].reshape(_B * _HW, _CIN)
    w_idx = lax.rem(lax.broadcasted_iota(jnp.int32, (_B * _HW, 1), 0), _W)
    xl = jnp.where(w_idx == _W - 1, jnp.bfloat16(0), xb)
    xr = jnp.where(w_idx == 0, jnp.bfloat16(0), xb)
    taps = (xl, xb, xr)

    for b in range(_B):
        base = b * _HW
        zeros = jnp.zeros((48, _K), jnp.bfloat16)
        xc_ref[s, pl.ds(base, 48), :] = zeros
        xc_ref[s, pl.ds(base + _HW - 48, 48), :] = zeros
        for dy in range(_KH):
            for dx in range(_KW):
                off = (dy - 1) * _W + (dx - 1)
                lo = max(0, -off)
                hi = min(_HW, _HW - off)
                k0 = (dy * _KW + dx) * _CIN
                xc_ref[s, pl.ds(base + lo, hi - lo), k0:k0 + _CIN] = (
                    taps[dx][base + lo + off:base + hi + off])

    p = jnp.dot(xc_ref[1 - s], w_ref[...], preferred_element_type=jnp.float32)
    o_ref[...] = p.reshape(_B, _HW, _COUT)


def kernel(x, weight):
    n = x.shape[0]
    n_steps = n // _B
    scale = 1.0 / float(np.sqrt(np.prod(weight.shape[1:])))
    w_t = jnp.transpose(weight, (2, 3, 1, 0)).reshape(_K, _COUT)
    w_t = (w_t * scale).astype(jnp.bfloat16)
    x_nhwc = jnp.transpose(x, (0, 2, 3, 1)).reshape(n, _HW, _CIN)
    x_nhwc = x_nhwc.astype(jnp.bfloat16)

    out = pl.pallas_call(
        _conv_body,
        out_shape=jax.ShapeDtypeStruct((n, _HW, _COUT), jnp.float32),
        grid=(n_steps + 1,),
        in_specs=[
            pl.BlockSpec((_B, _HW, _CIN),
                         lambda i: (jnp.minimum(i, n_steps - 1), 0, 0)),
            pl.BlockSpec((_K, _COUT), lambda i: (0, 0)),
        ],
        out_specs=pl.BlockSpec((_B, _HW, _COUT),
                               lambda i: (jnp.maximum(i - 1, 0), 0, 0)),
        scratch_shapes=[pltpu.VMEM((2, _B * _HW, _K), jnp.bfloat16)],
        compiler_params=pltpu.CompilerParams(
            dimension_semantics=("arbitrary",),
            vmem_limit_bytes=64 * 1024 * 1024),
    )(x_nhwc, w_t)
    out = out.reshape(n, _H, _W, _COUT)
    return jnp.transpose(out, (0, 3, 1, 2))


# speedup vs baseline: 1.6707x; 1.1858x over previous
import numpy as np
import jax
import jax.numpy as jnp
from jax import lax
from jax.experimental import pallas as pl
from jax.experimental.pallas import tpu as pltpu

_H = 32
_W = 32
_CIN = 128
_COUT = 256
_KH = 3
_KW = 3
_HW = _H * _W
_K = _KH * _KW * _CIN
_B = 8


def _conv_body(x_ref, w_ref, o_ref, xc_ref):
    xb = x_ref[...].reshape(_B * _HW, _CIN)
    w_idx = lax.rem(lax.broadcasted_iota(jnp.int32, (_B * _HW, 1), 0), _W)
    xl = jnp.where(w_idx == _W - 1, jnp.bfloat16(0), xb)
    xr = jnp.where(w_idx == 0, jnp.bfloat16(0), xb)
    taps = (xl, xb, xr)

    def build(b):
        base = b * _HW
        zeros = jnp.zeros((48, _K), jnp.bfloat16)
        xc_ref[pl.ds(base, 48), :] = zeros
        xc_ref[pl.ds(base + _HW - 48, 48), :] = zeros
        for dy in range(_KH):
            for dx in range(_KW):
                off = (dy - 1) * _W + (dx - 1)
                lo = max(0, -off)
                hi = min(_HW, _HW - off)
                k0 = (dy * _KW + dx) * _CIN
                xc_ref[pl.ds(base + lo, hi - lo), k0:k0 + _CIN] = (
                    taps[dx][base + lo + off:base + hi + off])

    def matmul(b):
        p = jnp.dot(xc_ref[pl.ds(b * _HW, _HW), :], w_ref[...],
                    preferred_element_type=jnp.float32)
        o_ref[b] = p.reshape(_HW, _COUT)

    build(0)
    for b in range(1, _B):
        build(b)
        matmul(b - 1)
    matmul(_B - 1)


def kernel(x, weight):
    n = x.shape[0]
    scale = 1.0 / float(np.sqrt(np.prod(weight.shape[1:])))
    w_t = jnp.transpose(weight, (2, 3, 1, 0)).reshape(_K, _COUT)
    w_t = (w_t * scale).astype(jnp.bfloat16)
    x_nhwc = jnp.transpose(x, (0, 2, 3, 1)).reshape(n, _HW, _CIN)
    x_nhwc = x_nhwc.astype(jnp.bfloat16)

    out = pl.pallas_call(
        _conv_body,
        out_shape=jax.ShapeDtypeStruct((n, _HW, _COUT), jnp.float32),
        grid=(n // _B,),
        in_specs=[
            pl.BlockSpec((_B, _HW, _CIN), lambda i: (i, 0, 0)),
            pl.BlockSpec((_K, _COUT), lambda i: (0, 0)),
        ],
        out_specs=pl.BlockSpec((_B, _HW, _COUT), lambda i: (i, 0, 0)),
        scratch_shapes=[pltpu.VMEM((_B * _HW, _K), jnp.bfloat16)],
        compiler_params=pltpu.CompilerParams(
            dimension_semantics=("parallel",),
            vmem_limit_bytes=64 * 1024 * 1024),
    )(x_nhwc, w_t)
    out = out.reshape(n, _H, _W, _COUT)
    return jnp.transpose(out, (0, 3, 1, 2))
